# Initial kernel scaffold; baseline (speedup 1.0000x reference)
import jax
import jax.numpy as jnp
from jax import lax
from jax.experimental import pallas as pl
from jax.experimental.pallas import tpu as pltpu
from jax.experimental.pallas import tpu_sc as plsc

B, S, D, V = 4096, 200, 64, 100000
NC, NS = 2, 16
NW = NC * NS
BPW = B // NW
HALF = S // 2


def _body(x_hbm, table_hbm, cls_hbm, pos_hbm, out_hbm,
          idx_v, buf, pos_v, sem):
    wid = lax.axis_index("s") * NC + lax.axis_index("c")
    base = wid * BPW

    pltpu.sync_copy(pos_hbm, pos_v)
    pltpu.sync_copy(cls_hbm, buf.at[0])

    @pl.loop(0, BPW)
    def _(b):
        row = base + b
        pltpu.sync_copy(x_hbm.at[row], idx_v)
        c0 = pltpu.async_copy(table_hbm.at[idx_v.at[0]],
                              buf.at[pl.ds(1, HALF)], sem)
        c1 = pltpu.async_copy(table_hbm.at[idx_v.at[1]],
                              buf.at[pl.ds(1 + HALF, HALF)], sem)
        c0.wait()
        c1.wait()

        @pl.loop(0, S)
        def _(r):
            for c in range(D // 16):
                sl = pl.ds(c * 16, 16)
                buf[1 + r, sl] = buf[1 + r, sl] + pos_v[r, sl]

        pltpu.sync_copy(buf, out_hbm.at[row])


def kernel(x, value_table, cls_token, pos_embedding):
    x2 = x.reshape(B, 2, HALF)
    run = pl.kernel(
        _body,
        out_type=jax.ShapeDtypeStruct((B, S + 1, D), jnp.float32),
        mesh=plsc.VectorSubcoreMesh(core_axis_name="c", subcore_axis_name="s"),
        scratch_types=[
            pltpu.VMEM((2, HALF), jnp.int32),
            pltpu.VMEM((S + 1, D), jnp.float32),
            pltpu.VMEM((S, D), jnp.float32),
            pltpu.SemaphoreType.DMA,
        ],
    )
    return run(x2, value_table, cls_token, pos_embedding)


# speedup vs baseline: 3.1943x; 3.1943x over previous
import jax
import jax.numpy as jnp
from jax import lax
from jax.experimental import pallas as pl
from jax.experimental.pallas import tpu as pltpu
from jax.experimental.pallas import tpu_sc as plsc

B, S, D, V = 4096, 200, 64, 100000
NC, NS = 2, 16
NW = NC * NS
BPW = B // NW
HALF = S // 2


def _body(x_hbm, table_hbm, cls_hbm, pos_hbm, out_hbm,
          idx_v, buf, pos_v, sem):
    wid = lax.axis_index("s") * NC + lax.axis_index("c")
    base = wid * BPW

    pltpu.sync_copy(pos_hbm, pos_v)
    pltpu.sync_copy(cls_hbm, buf.at[0])

    @pl.loop(0, BPW)
    def _(b):
        row = base + b
        pltpu.sync_copy(x_hbm.at[row], idx_v)
        c0 = pltpu.async_copy(table_hbm.at[idx_v.at[0]],
                              buf.at[pl.ds(1, HALF)], sem)
        c1 = pltpu.async_copy(table_hbm.at[idx_v.at[1]],
                              buf.at[pl.ds(1 + HALF, HALF)], sem)
        c0.wait()
        c1.wait()

        @pl.loop(0, S)
        def _(r):
            for c in range(D // 16):
                sl = pl.ds(c * 16, 16)
                buf[1 + r, sl] = buf[1 + r, sl] + pos_v[r, sl]

        pltpu.sync_copy(buf, out_hbm.at[row])


def kernel(x, value_table, cls_token, pos_embedding):
    x2 = x.reshape(B, 2, HALF)
    run = pl.kernel(
        _body,
        out_type=jax.ShapeDtypeStruct((B, S + 1, D), jnp.float32),
        mesh=plsc.VectorSubcoreMesh(core_axis_name="c", subcore_axis_name="s"),
        scratch_types=[
            pltpu.VMEM((2, HALF), jnp.int32),
            pltpu.VMEM((S + 1, D), jnp.float32),
            pltpu.VMEM((S, D), jnp.float32),
            pltpu.SemaphoreType.DMA,
        ],
        compiler_params=pltpu.CompilerParams(use_tc_tiling_on_sc=False),
    )
    return run(x2, value_table, cls_token, pos_embedding)


# speedup vs baseline: 4.1382x; 1.2955x over previous
import jax
import jax.numpy as jnp
from jax import lax
from jax.experimental import pallas as pl
from jax.experimental.pallas import tpu as pltpu
from jax.experimental.pallas import tpu_sc as plsc

B, S, D, V = 4096, 200, 64, 100000
NC, NS = 2, 16
NW = NC * NS
BPW = B // NW
HALF = S // 2
NBUF = 4


def _body(x_hbm, table_hbm, cls_hbm, pos_hbm, out_hbm,
          idx_v, bufs, pos_v, gsems, osems):
    wid = lax.axis_index("s") * NC + lax.axis_index("c")
    base = wid * BPW

    pltpu.sync_copy(pos_hbm, pos_v)
    pltpu.sync_copy(x_hbm.at[pl.ds(base * 2, BPW * 2)], idx_v)
    for k in range(NBUF):
        pltpu.sync_copy(cls_hbm, bufs.at[k, 0])

    def fire_gather(b, k):
        pltpu.async_copy(table_hbm.at[idx_v.at[2 * b]],
                         bufs.at[k, pl.ds(1, HALF)], gsems[k])
        pltpu.async_copy(table_hbm.at[idx_v.at[2 * b + 1]],
                         bufs.at[k, pl.ds(1 + HALF, HALF)], gsems[k])

    def wait_gather(k):
        d = pltpu.make_async_copy(table_hbm.at[idx_v.at[0]],
                                  bufs.at[k, pl.ds(1, HALF)], gsems[k])
        d.wait()
        d.wait()

    def wait_write(k):
        pltpu.make_async_copy(bufs.at[k], out_hbm.at[base], osems[k]).wait()

    def add_pos(k):
        @pl.loop(0, S, unroll=4)
        def _(r):
            for c in range(D // 16):
                sl = pl.ds(c * 16, 16)
                plsc.addupdate(bufs.at[k, 1 + r, sl], pos_v[r, sl])

    def fire_write(b, k):
        pltpu.async_copy(bufs.at[k], out_hbm.at[base + b], osems[k])

    for k in range(NBUF - 1):
        fire_gather(k, k)
    for b in range(NBUF):
        if b > 0:
            wait_write((b + NBUF - 1) % NBUF)
        fire_gather(b + NBUF - 1, (b + NBUF - 1) % NBUF)
        wait_gather(b)
        add_pos(b)
        fire_write(b, b)

    @pl.loop(NBUF, BPW, step=NBUF)
    def _(b0):
        for k in range(NBUF):
            b = b0 + k
            fb = b + NBUF - 1
            fk = (k + NBUF - 1) % NBUF

            @pl.when(fb < BPW)
            def _():
                wait_write(fk)
                fire_gather(fb, fk)

            wait_gather(k)
            add_pos(k)
            fire_write(b, k)

    for k in range(NBUF):
        wait_write(k)


def kernel(x, value_table, cls_token, pos_embedding):
    x2 = x.reshape(B * 2, HALF)
    run = pl.kernel(
        _body,
        out_type=jax.ShapeDtypeStruct((B, S + 1, D), jnp.float32),
        mesh=plsc.VectorSubcoreMesh(core_axis_name="c", subcore_axis_name="s"),
        scratch_types=[
            pltpu.VMEM((BPW * 2, HALF), jnp.int32),
            pltpu.VMEM((NBUF, S + 1, D), jnp.float32),
            pltpu.VMEM((S, D), jnp.float32),
            [pltpu.SemaphoreType.DMA] * NBUF,
            [pltpu.SemaphoreType.DMA] * NBUF,
        ],
        compiler_params=pltpu.CompilerParams(use_tc_tiling_on_sc=False),
    )
    return run(x2, value_table, cls_token, pos_embedding)
